# baseline (speedup 1.0000x reference)
import functools

import jax
import jax.numpy as jnp
from jax import lax
from jax.experimental import pallas as pl
from jax.experimental.pallas import tpu as pltpu
from jax.experimental.pallas import tpu_sc as plsc

NUM_GRAPHS = 128
K = 768
_SHARE0 = 1 / 3


def _fill_rows(ref, nrows, value):
    def body(i, _):
        ref[i, :] = jnp.full((16,), value, jnp.float32)
        return _
    lax.fori_loop(0, nrows, body, None)


def _fill_flat(ref, nvec, value):
    def body(i, _):
        ref[pl.ds(i * 16, 16)] = jnp.full((16,), value, jnp.float32)
        return _
    lax.fori_loop(0, nvec, body, None)


def _zero_shared(acc, zeros_buf, zlen, start, count, align=8):
    done = 0
    while done < count:
        step = min(zlen, count - done)
        pltpu.sync_copy(zeros_buf.at[pl.ds(0, step)],
                        acc.at[pl.ds(pl.multiple_of(start + done, align),
                                     step)])
        done += step


def _make_deg_kernel(epad, nacc, nc, ns):
    nw = nc * ns
    ep = epad // nw
    nchunk = ep // K
    per_tile = nacc // ns
    mesh = plsc.VectorSubcoreMesh(core_axis_name="c", subcore_axis_name="s")

    @functools.partial(
        pl.kernel, mesh=mesh,
        out_type=jax.ShapeDtypeStruct((nc, nacc), jnp.float32),
        scratch_types=[
            pltpu.VMEM((2, K), jnp.int32),
            pltpu.VMEM((K,), jnp.float32),
            pltpu.VMEM((K,), jnp.float32),
            pltpu.VMEM_SHARED((nacc,), jnp.float32),
        ],
        compiler_params=pltpu.CompilerParams(use_tc_tiling_on_sc=False),
    )
    def deg_kernel(ei_hbm, out_hbm, ebuf, ones, zeros, acc):
        c = lax.axis_index("c")
        s = lax.axis_index("s")
        _fill_flat(ones, K // 16, 1.0)
        _fill_flat(zeros, K // 16, 0.0)
        _zero_shared(acc, zeros, K, s * per_tile, per_tile, align=128)
        plsc.subcore_barrier()

        base0 = (c * ns + s) * ep

        def chunk(g, _):
            base = pl.multiple_of(base0 + g * K, 256)
            pltpu.sync_copy(ei_hbm.at[:, pl.ds(base, K)], ebuf)
            pltpu.sync_copy(ones, acc.at[ebuf.at[1]], add=True)
            return _

        lax.fori_loop(0, nchunk, chunk, None)
        plsc.subcore_barrier()
        d0 = pl.multiple_of(s * per_tile, 128)
        pltpu.sync_copy(acc.at[pl.ds(d0, per_tile)],
                        out_hbm.at[c].at[pl.ds(d0, per_tile)])

    return deg_kernel


def _make_edge_agg_kernel(n, epad, nc, ns, feat, dtype, share0):
    nacc = ((n + ns * 8 - 1) // (ns * 8)) * (ns * 8)
    per_tile = nacc // ns
    pairs_total = epad // (2 * K * ns)
    npair0 = max(1, min(pairs_total - 1, round(share0 * pairs_total)))
    npair1 = pairs_total - npair0
    ep0 = 2 * K * npair0
    ep1 = 2 * K * npair1
    mesh = plsc.VectorSubcoreMesh(core_axis_name="c", subcore_axis_name="s")

    @functools.partial(
        pl.kernel, mesh=mesh,
        out_type=jax.ShapeDtypeStruct((nc, nacc, feat), dtype),
        scratch_types=[
            pltpu.VMEM((2, 2, K), jnp.int32),
            pltpu.VMEM((2, K, feat), dtype),
            pltpu.VMEM_SHARED((nacc, feat), dtype),
            pltpu.SemaphoreType.DMA,
            pltpu.SemaphoreType.DMA,
            pltpu.SemaphoreType.DMA,
            pltpu.SemaphoreType.DMA,
        ],
        compiler_params=pltpu.CompilerParams(use_tc_tiling_on_sc=False),
    )
    def agg_kernel(tab_hbm, ei_hbm, out_hbm, ebuf, rows, acc,
                   gs0, gs1, is0, is1):
        c = lax.axis_index("c")
        s = lax.axis_index("s")
        gsem = (gs0, gs1)
        isem = (is0, is1)
        def zrow(i, _):
            rows[0, i, :] = jnp.zeros((feat,), dtype)
            return _
        lax.fori_loop(0, K, zrow, None)
        _zero_shared(acc, rows.at[0], K, s * per_tile, per_tile)
        plsc.subcore_barrier()

        npair = jnp.where(c == 0, npair0, npair1)
        base0 = jnp.where(c == 0, s * ep0, ns * ep0 + s * ep1)

        def idx_copy(b, q):
            base = pl.multiple_of(base0 + q * K, 256)
            return pltpu.make_async_copy(
                ei_hbm.at[:, pl.ds(base, K)], ebuf.at[b], isem[b])

        def gather_copy(b):
            return pltpu.make_async_copy(
                tab_hbm.at[ebuf.at[b].at[0]], rows.at[b], gsem[b])

        def scatter(b):
            pltpu.sync_copy(rows.at[b], acc.at[ebuf.at[b].at[1]], add=True)

        idx_copy(0, 0).start()
        idx_copy(0, 0).wait()
        gather_copy(0).start()
        idx_copy(1, 1).start()

        def pair(t, _):
            idx_copy(1, 2 * t + 1).wait()
            gather_copy(1).start()
            gather_copy(0).wait()
            scatter(0)

            @pl.when(t < npair - 1)
            def _():
                idx_copy(0, 2 * t + 2).start()

            gather_copy(1).wait()

            @pl.when(t < npair - 1)
            def _():
                idx_copy(0, 2 * t + 2).wait()
                gather_copy(0).start()

            scatter(1)

            @pl.when(t < npair - 1)
            def _():
                idx_copy(1, 2 * t + 3).start()

            return _

        lax.fori_loop(0, npair, pair, None)
        plsc.subcore_barrier()
        r0 = pl.multiple_of(s * per_tile, 8)
        pltpu.sync_copy(acc.at[pl.ds(r0, per_tile)],
                        out_hbm.at[c].at[pl.ds(r0, per_tile)])

    return agg_kernel


def _prep_body(deg_ref, x_ref, dinv_ref, u_ref):
    deg = deg_ref[0, :, :] + deg_ref[1, :, :] + 1.0
    dinv = lax.rsqrt(deg)
    dinv_ref[...] = dinv
    u_ref[...] = x_ref[...] * dinv


def _mid_body(agg_ref, u_ref, dinv_ref, w1_ref, b1_ref, w2_ref, z_ref):
    dinv = dinv_ref[...]
    ax = (agg_ref[0] + agg_ref[1] + u_ref[...]) * dinv
    h1 = jnp.maximum(
        jnp.dot(ax, w1_ref[...], preferred_element_type=jnp.float32)
        + b1_ref[...], 0.0)
    z2 = jnp.dot(h1, w2_ref[...], preferred_element_type=jnp.float32) * dinv
    z_ref[...] = z2.astype(jnp.bfloat16)


def _final_body(nblk, agg2_ref, z_ref, dinv_ref, batch_ref, b2_ref,
                fc1w_ref, fc1b_ref, fc2w_ref, fc2b_ref, out_ref, acc_ref):
    i = pl.program_id(0)
    r = batch_ref.shape[0]
    agg2 = (agg2_ref[0].astype(jnp.float32)
            + agg2_ref[1].astype(jnp.float32))
    z2 = z_ref[...].astype(jnp.float32)
    out2 = (agg2 + z2) * dinv_ref[...]
    out2a = jnp.concatenate([out2, jnp.ones((r, 1), jnp.float32)], axis=1)
    ids = lax.broadcasted_iota(jnp.int32, (r, NUM_GRAPHS), 1)
    oh = (ids == batch_ref[...]).astype(jnp.float32)
    contrib = lax.dot_general(oh, out2a, (((0,), (0,)), ((), ())),
                              preferred_element_type=jnp.float32)

    @pl.when(i == 0)
    def _():
        acc_ref[...] = contrib

    @pl.when(i > 0)
    def _():
        acc_ref[...] += contrib

    @pl.when(i == nblk - 1)
    def _():
        acc = acc_ref[...]
        g = (acc[:, :32] / jnp.maximum(acc[:, 32:33], 1.0)) + b2_ref[...]
        h = jnp.maximum(
            jnp.dot(g, fc1w_ref[...], preferred_element_type=jnp.float32)
            + fc1b_ref[...], 0.0)
        out_ref[...] = (
            jnp.dot(h, fc2w_ref[...], preferred_element_type=jnp.float32)
            + fc2b_ref[...])


def kernel(x, edge_index, batch, W1, b1, W2, b2, fc1_W, fc1_b, fc2_W, fc2_b):
    n, f = x.shape
    e = edge_index.shape[1]
    nc, ns = 2, 16
    nw = nc * ns

    step = 2 * nw * K
    epad = ((e + step - 1) // step) * step
    pad = epad - e
    eip = jnp.concatenate(
        [edge_index,
         jnp.stack([jnp.zeros((pad,), jnp.int32),
                    jnp.full((pad,), n, jnp.int32)])], axis=1)
    x16 = jnp.pad(x, ((0, 0), (0, 16 - f)))
    w1p = jnp.pad(W1, ((0, 16 - f), (0, 0)))
    nacc = ((n + ns * 128 - 1) // (ns * 128)) * (ns * 128)

    deg2 = _make_deg_kernel(epad, nacc, nc, ns)(eip)
    naccr = ((n + ns * 8 - 1) // (ns * 8)) * (ns * 8)
    deg3 = deg2.reshape(nc, nacc, 1)

    rblk = 5000
    nblk = n // rblk
    dinv, u = pl.pallas_call(
        _prep_body,
        grid=(nblk,),
        in_specs=[
            pl.BlockSpec((nc, rblk, 1), lambda i: (0, i, 0)),
            pl.BlockSpec((rblk, 16), lambda i: (i, 0)),
        ],
        out_specs=[
            pl.BlockSpec((rblk, 1), lambda i: (i, 0)),
            pl.BlockSpec((rblk, 16), lambda i: (i, 0)),
        ],
        out_shape=[
            jax.ShapeDtypeStruct((n, 1), jnp.float32),
            jax.ShapeDtypeStruct((n, 16), jnp.float32),
        ],
    )(deg3, x16)

    agg = _make_edge_agg_kernel(n, epad, nc, ns, 16, jnp.float32, _SHARE0)(u, eip)

    z = pl.pallas_call(
        _mid_body,
        grid=(nblk,),
        in_specs=[
            pl.BlockSpec((nc, rblk, 16), lambda i: (0, i, 0)),
            pl.BlockSpec((rblk, 16), lambda i: (i, 0)),
            pl.BlockSpec((rblk, 1), lambda i: (i, 0)),
            pl.BlockSpec((16, 64), lambda i: (0, 0)),
            pl.BlockSpec((1, 64), lambda i: (0, 0)),
            pl.BlockSpec((64, 32), lambda i: (0, 0)),
        ],
        out_specs=pl.BlockSpec((rblk, 32), lambda i: (i, 0)),
        out_shape=jax.ShapeDtypeStruct((n, 32), jnp.bfloat16),
    )(agg, u, dinv, w1p, b1.reshape(1, 64), W2)

    agg2 = _make_edge_agg_kernel(n, epad, nc, ns, 32, jnp.bfloat16, _SHARE0)(z, eip)

    out = pl.pallas_call(
        functools.partial(_final_body, nblk),
        grid=(nblk,),
        in_specs=[
            pl.BlockSpec((nc, rblk, 32), lambda i: (0, i, 0)),
            pl.BlockSpec((rblk, 32), lambda i: (i, 0)),
            pl.BlockSpec((rblk, 1), lambda i: (i, 0)),
            pl.BlockSpec((rblk, 1), lambda i: (i, 0)),
            pl.BlockSpec((1, 32), lambda i: (0, 0)),
            pl.BlockSpec((32, 32), lambda i: (0, 0)),
            pl.BlockSpec((1, 32), lambda i: (0, 0)),
            pl.BlockSpec((32, 32), lambda i: (0, 0)),
            pl.BlockSpec((1, 32), lambda i: (0, 0)),
        ],
        out_specs=pl.BlockSpec((NUM_GRAPHS, 32), lambda i: (0, 0)),
        out_shape=jax.ShapeDtypeStruct((NUM_GRAPHS, 32), jnp.float32),
        scratch_shapes=[
            pltpu.VMEM((NUM_GRAPHS, 33), jnp.float32),
        ],
    )(agg2, z, dinv, batch.reshape(n, 1), b2.reshape(1, 32),
      fc1_W, fc1_b.reshape(1, 32), fc2_W, fc2_b.reshape(1, 32))
    return out


# speedup vs baseline: 1.0967x; 1.0967x over previous
import functools

import jax
import jax.numpy as jnp
from jax import lax
from jax.experimental import pallas as pl
from jax.experimental.pallas import tpu as pltpu
from jax.experimental.pallas import tpu_sc as plsc

NUM_GRAPHS = 128
K = 768
_SHARE0 = 2 / 3


def _fill_rows(ref, nrows, value):
    def body(i, _):
        ref[i, :] = jnp.full((16,), value, jnp.float32)
        return _
    lax.fori_loop(0, nrows, body, None)


def _fill_flat(ref, nvec, value):
    def body(i, _):
        ref[pl.ds(i * 16, 16)] = jnp.full((16,), value, jnp.float32)
        return _
    lax.fori_loop(0, nvec, body, None)


def _zero_shared(acc, zeros_buf, zlen, start, count, align=8):
    done = 0
    while done < count:
        step = min(zlen, count - done)
        pltpu.sync_copy(zeros_buf.at[pl.ds(0, step)],
                        acc.at[pl.ds(pl.multiple_of(start + done, align),
                                     step)])
        done += step


def _make_deg_kernel(epad, nacc, nc, ns):
    nw = nc * ns
    ep = epad // nw
    nchunk = ep // K
    per_tile = nacc // ns
    mesh = plsc.VectorSubcoreMesh(core_axis_name="c", subcore_axis_name="s")

    @functools.partial(
        pl.kernel, mesh=mesh,
        out_type=jax.ShapeDtypeStruct((nc, nacc), jnp.float32),
        scratch_types=[
            pltpu.VMEM((2, K), jnp.int32),
            pltpu.VMEM((K,), jnp.float32),
            pltpu.VMEM((K,), jnp.float32),
            pltpu.VMEM_SHARED((nacc,), jnp.float32),
        ],
        compiler_params=pltpu.CompilerParams(use_tc_tiling_on_sc=False),
    )
    def deg_kernel(ei_hbm, out_hbm, ebuf, ones, zeros, acc):
        c = lax.axis_index("c")
        s = lax.axis_index("s")
        _fill_flat(ones, K // 16, 1.0)
        _fill_flat(zeros, K // 16, 0.0)
        _zero_shared(acc, zeros, K, s * per_tile, per_tile, align=128)
        plsc.subcore_barrier()

        base0 = (c * ns + s) * ep

        def chunk(g, _):
            base = pl.multiple_of(base0 + g * K, 256)
            pltpu.sync_copy(ei_hbm.at[:, pl.ds(base, K)], ebuf)
            pltpu.sync_copy(ones, acc.at[ebuf.at[1]], add=True)
            return _

        lax.fori_loop(0, nchunk, chunk, None)
        plsc.subcore_barrier()
        d0 = pl.multiple_of(s * per_tile, 128)
        pltpu.sync_copy(acc.at[pl.ds(d0, per_tile)],
                        out_hbm.at[c].at[pl.ds(d0, per_tile)])

    return deg_kernel


def _make_edge_agg_kernel(n, epad, nc, ns, feat, dtype, share0):
    nacc = ((n + ns * 8 - 1) // (ns * 8)) * (ns * 8)
    per_tile = nacc // ns
    pairs_total = epad // (2 * K * ns)
    npair0 = max(1, min(pairs_total - 1, round(share0 * pairs_total)))
    npair1 = pairs_total - npair0
    ep0 = 2 * K * npair0
    ep1 = 2 * K * npair1
    mesh = plsc.VectorSubcoreMesh(core_axis_name="c", subcore_axis_name="s")

    @functools.partial(
        pl.kernel, mesh=mesh,
        out_type=jax.ShapeDtypeStruct((nc, nacc, feat), dtype),
        scratch_types=[
            pltpu.VMEM((2, 2, K), jnp.int32),
            pltpu.VMEM((2, K, feat), dtype),
            pltpu.VMEM_SHARED((nacc, feat), dtype),
            pltpu.SemaphoreType.DMA,
            pltpu.SemaphoreType.DMA,
            pltpu.SemaphoreType.DMA,
            pltpu.SemaphoreType.DMA,
        ],
        compiler_params=pltpu.CompilerParams(use_tc_tiling_on_sc=False),
    )
    def agg_kernel(tab_hbm, ei_hbm, out_hbm, ebuf, rows, acc,
                   gs0, gs1, is0, is1):
        c = lax.axis_index("c")
        s = lax.axis_index("s")
        gsem = (gs0, gs1)
        isem = (is0, is1)
        def zrow(i, _):
            rows[0, i, :] = jnp.zeros((feat,), dtype)
            return _
        lax.fori_loop(0, K, zrow, None)
        _zero_shared(acc, rows.at[0], K, s * per_tile, per_tile)
        plsc.subcore_barrier()

        npair = jnp.where(c == 0, npair0, npair1)
        base0 = jnp.where(c == 0, s * ep0, ns * ep0 + s * ep1)

        def idx_copy(b, q):
            base = pl.multiple_of(base0 + q * K, 256)
            return pltpu.make_async_copy(
                ei_hbm.at[:, pl.ds(base, K)], ebuf.at[b], isem[b])

        def gather_copy(b):
            return pltpu.make_async_copy(
                tab_hbm.at[ebuf.at[b].at[0]], rows.at[b], gsem[b])

        def scatter(b):
            pltpu.sync_copy(rows.at[b], acc.at[ebuf.at[b].at[1]], add=True)

        idx_copy(0, 0).start()
        idx_copy(0, 0).wait()
        gather_copy(0).start()
        idx_copy(1, 1).start()

        def pair(t, _):
            idx_copy(1, 2 * t + 1).wait()
            gather_copy(1).start()
            gather_copy(0).wait()
            scatter(0)

            @pl.when(t < npair - 1)
            def _():
                idx_copy(0, 2 * t + 2).start()

            gather_copy(1).wait()

            @pl.when(t < npair - 1)
            def _():
                idx_copy(0, 2 * t + 2).wait()
                gather_copy(0).start()

            scatter(1)

            @pl.when(t < npair - 1)
            def _():
                idx_copy(1, 2 * t + 3).start()

            return _

        lax.fori_loop(0, npair, pair, None)
        plsc.subcore_barrier()
        r0 = pl.multiple_of(s * per_tile, 8)
        pltpu.sync_copy(acc.at[pl.ds(r0, per_tile)],
                        out_hbm.at[c].at[pl.ds(r0, per_tile)])

    return agg_kernel


def _prep_body(deg_ref, x_ref, dinv_ref, u_ref):
    deg = deg_ref[0, :, :] + deg_ref[1, :, :] + 1.0
    dinv = lax.rsqrt(deg)
    dinv_ref[...] = dinv
    u_ref[...] = x_ref[...] * dinv


def _mid_body(agg_ref, u_ref, dinv_ref, w1_ref, b1_ref, w2_ref, z_ref):
    dinv = dinv_ref[...]
    ax = (agg_ref[0] + agg_ref[1] + u_ref[...]) * dinv
    h1 = jnp.maximum(
        jnp.dot(ax, w1_ref[...], preferred_element_type=jnp.float32)
        + b1_ref[...], 0.0)
    z2 = jnp.dot(h1, w2_ref[...], preferred_element_type=jnp.float32) * dinv
    z_ref[...] = z2.astype(jnp.bfloat16)


def _final_body(nblk, agg2_ref, z_ref, dinv_ref, batch_ref, b2_ref,
                fc1w_ref, fc1b_ref, fc2w_ref, fc2b_ref, out_ref, acc_ref):
    i = pl.program_id(0)
    r = batch_ref.shape[0]
    agg2 = (agg2_ref[0].astype(jnp.float32)
            + agg2_ref[1].astype(jnp.float32))
    z2 = z_ref[...].astype(jnp.float32)
    out2 = (agg2 + z2) * dinv_ref[...]
    out2a = jnp.concatenate([out2, jnp.ones((r, 1), jnp.float32)], axis=1)
    ids = lax.broadcasted_iota(jnp.int32, (r, NUM_GRAPHS), 1)
    oh = (ids == batch_ref[...]).astype(jnp.float32)
    contrib = lax.dot_general(oh, out2a, (((0,), (0,)), ((), ())),
                              preferred_element_type=jnp.float32)

    @pl.when(i == 0)
    def _():
        acc_ref[...] = contrib

    @pl.when(i > 0)
    def _():
        acc_ref[...] += contrib

    @pl.when(i == nblk - 1)
    def _():
        acc = acc_ref[...]
        g = (acc[:, :32] / jnp.maximum(acc[:, 32:33], 1.0)) + b2_ref[...]
        h = jnp.maximum(
            jnp.dot(g, fc1w_ref[...], preferred_element_type=jnp.float32)
            + fc1b_ref[...], 0.0)
        out_ref[...] = (
            jnp.dot(h, fc2w_ref[...], preferred_element_type=jnp.float32)
            + fc2b_ref[...])


def kernel(x, edge_index, batch, W1, b1, W2, b2, fc1_W, fc1_b, fc2_W, fc2_b):
    n, f = x.shape
    e = edge_index.shape[1]
    nc, ns = 2, 16
    nw = nc * ns

    step = 2 * nw * K
    epad = ((e + step - 1) // step) * step
    pad = epad - e
    eip = jnp.concatenate(
        [edge_index,
         jnp.stack([jnp.zeros((pad,), jnp.int32),
                    jnp.full((pad,), n, jnp.int32)])], axis=1)
    x16 = jnp.pad(x, ((0, 0), (0, 16 - f)))
    w1p = jnp.pad(W1, ((0, 16 - f), (0, 0)))
    nacc = ((n + ns * 128 - 1) // (ns * 128)) * (ns * 128)

    deg2 = _make_deg_kernel(epad, nacc, nc, ns)(eip)
    naccr = ((n + ns * 8 - 1) // (ns * 8)) * (ns * 8)
    deg3 = deg2.reshape(nc, nacc, 1)

    rblk = 5000
    nblk = n // rblk
    dinv, u = pl.pallas_call(
        _prep_body,
        grid=(nblk,),
        in_specs=[
            pl.BlockSpec((nc, rblk, 1), lambda i: (0, i, 0)),
            pl.BlockSpec((rblk, 16), lambda i: (i, 0)),
        ],
        out_specs=[
            pl.BlockSpec((rblk, 1), lambda i: (i, 0)),
            pl.BlockSpec((rblk, 16), lambda i: (i, 0)),
        ],
        out_shape=[
            jax.ShapeDtypeStruct((n, 1), jnp.float32),
            jax.ShapeDtypeStruct((n, 16), jnp.float32),
        ],
    )(deg3, x16)

    agg = _make_edge_agg_kernel(n, epad, nc, ns, 16, jnp.float32, _SHARE0)(u, eip)

    z = pl.pallas_call(
        _mid_body,
        grid=(nblk,),
        in_specs=[
            pl.BlockSpec((nc, rblk, 16), lambda i: (0, i, 0)),
            pl.BlockSpec((rblk, 16), lambda i: (i, 0)),
            pl.BlockSpec((rblk, 1), lambda i: (i, 0)),
            pl.BlockSpec((16, 64), lambda i: (0, 0)),
            pl.BlockSpec((1, 64), lambda i: (0, 0)),
            pl.BlockSpec((64, 32), lambda i: (0, 0)),
        ],
        out_specs=pl.BlockSpec((rblk, 32), lambda i: (i, 0)),
        out_shape=jax.ShapeDtypeStruct((n, 32), jnp.bfloat16),
    )(agg, u, dinv, w1p, b1.reshape(1, 64), W2)

    agg2 = _make_edge_agg_kernel(n, epad, nc, ns, 32, jnp.bfloat16, _SHARE0)(z, eip)

    out = pl.pallas_call(
        functools.partial(_final_body, nblk),
        grid=(nblk,),
        in_specs=[
            pl.BlockSpec((nc, rblk, 32), lambda i: (0, i, 0)),
            pl.BlockSpec((rblk, 32), lambda i: (i, 0)),
            pl.BlockSpec((rblk, 1), lambda i: (i, 0)),
            pl.BlockSpec((rblk, 1), lambda i: (i, 0)),
            pl.BlockSpec((1, 32), lambda i: (0, 0)),
            pl.BlockSpec((32, 32), lambda i: (0, 0)),
            pl.BlockSpec((1, 32), lambda i: (0, 0)),
            pl.BlockSpec((32, 32), lambda i: (0, 0)),
            pl.BlockSpec((1, 32), lambda i: (0, 0)),
        ],
        out_specs=pl.BlockSpec((NUM_GRAPHS, 32), lambda i: (0, 0)),
        out_shape=jax.ShapeDtypeStruct((NUM_GRAPHS, 32), jnp.float32),
        scratch_shapes=[
            pltpu.VMEM((NUM_GRAPHS, 33), jnp.float32),
        ],
    )(agg2, z, dinv, batch.reshape(n, 1), b2.reshape(1, 32),
      fc1_W, fc1_b.reshape(1, 32), fc2_W, fc2_b.reshape(1, 32))
    return out


# speedup vs baseline: 1.1283x; 1.0288x over previous
import functools

import jax
import jax.numpy as jnp
from jax import lax
from jax.experimental import pallas as pl
from jax.experimental.pallas import tpu as pltpu
from jax.experimental.pallas import tpu_sc as plsc

NUM_GRAPHS = 128
K = 768
_SHARE0 = 2 / 3


def _fill_rows(ref, nrows, value):
    def body(i, _):
        ref[i, :] = jnp.full((16,), value, jnp.float32)
        return _
    lax.fori_loop(0, nrows, body, None)


def _fill_flat(ref, nvec, value):
    def body(i, _):
        ref[pl.ds(i * 16, 16)] = jnp.full((16,), value, jnp.float32)
        return _
    lax.fori_loop(0, nvec, body, None)


def _zero_shared(acc, zeros_buf, zlen, start, count, align=8):
    done = 0
    while done < count:
        step = min(zlen, count - done)
        pltpu.sync_copy(zeros_buf.at[pl.ds(0, step)],
                        acc.at[pl.ds(pl.multiple_of(start + done, align),
                                     step)])
        done += step


def _make_deg_kernel(epad, nacc, nc, ns):
    nw = nc * ns
    ep = epad // nw
    nchunk = ep // K
    per_tile = nacc // ns
    mesh = plsc.VectorSubcoreMesh(core_axis_name="c", subcore_axis_name="s")

    @functools.partial(
        pl.kernel, mesh=mesh,
        out_type=jax.ShapeDtypeStruct((nc, nacc), jnp.float32),
        scratch_types=[
            pltpu.VMEM((2, K), jnp.int32),
            pltpu.VMEM((K,), jnp.float32),
            pltpu.VMEM((K,), jnp.float32),
            pltpu.VMEM_SHARED((nacc,), jnp.float32),
        ],
        compiler_params=pltpu.CompilerParams(use_tc_tiling_on_sc=False),
    )
    def deg_kernel(ei_hbm, out_hbm, ebuf, ones, zeros, acc):
        c = lax.axis_index("c")
        s = lax.axis_index("s")
        _fill_flat(ones, K // 16, 1.0)
        _fill_flat(zeros, K // 16, 0.0)
        _zero_shared(acc, zeros, K, s * per_tile, per_tile, align=128)
        plsc.subcore_barrier()

        base0 = (c * ns + s) * ep

        def chunk(g, _):
            base = pl.multiple_of(base0 + g * K, 256)
            pltpu.sync_copy(ei_hbm.at[:, pl.ds(base, K)], ebuf)
            pltpu.sync_copy(ones, acc.at[ebuf.at[1]], add=True)
            return _

        lax.fori_loop(0, nchunk, chunk, None)
        plsc.subcore_barrier()
        d0 = pl.multiple_of(s * per_tile, 128)
        pltpu.sync_copy(acc.at[pl.ds(d0, per_tile)],
                        out_hbm.at[c].at[pl.ds(d0, per_tile)])

    return deg_kernel


def _make_edge_agg_kernel(n, epad, nc, ns, feat, dtype, share0, k):
    nacc = ((n + ns * 8 - 1) // (ns * 8)) * (ns * 8)
    per_tile = nacc // ns
    pairs_total = epad // (2 * k * ns)
    npair0 = max(1, min(pairs_total - 1, round(share0 * pairs_total)))
    npair1 = pairs_total - npair0
    ep0 = 2 * k * npair0
    ep1 = 2 * k * npair1
    mesh = plsc.VectorSubcoreMesh(core_axis_name="c", subcore_axis_name="s")

    @functools.partial(
        pl.kernel, mesh=mesh,
        out_type=jax.ShapeDtypeStruct((nc, nacc, feat), dtype),
        scratch_types=[
            pltpu.VMEM((2, 2, k), jnp.int32),
            pltpu.VMEM((2, k, feat), dtype),
            pltpu.VMEM_SHARED((nacc, feat), dtype),
            pltpu.SemaphoreType.DMA,
            pltpu.SemaphoreType.DMA,
            pltpu.SemaphoreType.DMA,
            pltpu.SemaphoreType.DMA,
        ],
        compiler_params=pltpu.CompilerParams(use_tc_tiling_on_sc=False),
    )
    def agg_kernel(tab_hbm, ei_hbm, out_hbm, ebuf, rows, acc,
                   gs0, gs1, is0, is1):
        c = lax.axis_index("c")
        s = lax.axis_index("s")
        gsem = (gs0, gs1)
        isem = (is0, is1)
        def zrow(i, _):
            rows[0, i, :] = jnp.zeros((feat,), dtype)
            return _
        lax.fori_loop(0, k, zrow, None)
        _zero_shared(acc, rows.at[0], k, s * per_tile, per_tile)
        plsc.subcore_barrier()

        npair = jnp.where(c == 0, npair0, npair1)
        base0 = jnp.where(c == 0, s * ep0, ns * ep0 + s * ep1)

        def idx_copy(b, q):
            base = pl.multiple_of(base0 + q * k, 256)
            return pltpu.make_async_copy(
                ei_hbm.at[:, pl.ds(base, k)], ebuf.at[b], isem[b])

        def gather_copy(b):
            return pltpu.make_async_copy(
                tab_hbm.at[ebuf.at[b].at[0]], rows.at[b], gsem[b])

        def scatter(b):
            pltpu.sync_copy(rows.at[b], acc.at[ebuf.at[b].at[1]], add=True)

        idx_copy(0, 0).start()
        idx_copy(0, 0).wait()
        gather_copy(0).start()
        idx_copy(1, 1).start()

        def pair(t, _):
            idx_copy(1, 2 * t + 1).wait()
            gather_copy(1).start()
            gather_copy(0).wait()
            scatter(0)

            @pl.when(t < npair - 1)
            def _():
                idx_copy(0, 2 * t + 2).start()

            gather_copy(1).wait()

            @pl.when(t < npair - 1)
            def _():
                idx_copy(0, 2 * t + 2).wait()
                gather_copy(0).start()

            scatter(1)

            @pl.when(t < npair - 1)
            def _():
                idx_copy(1, 2 * t + 3).start()

            return _

        lax.fori_loop(0, npair, pair, None)
        plsc.subcore_barrier()
        r0 = pl.multiple_of(s * per_tile, 8)
        pltpu.sync_copy(acc.at[pl.ds(r0, per_tile)],
                        out_hbm.at[c].at[pl.ds(r0, per_tile)])

    return agg_kernel


def _prep_body(deg_ref, x_ref, dinv_ref, u_ref):
    deg = deg_ref[0, :, :] + deg_ref[1, :, :] + 1.0
    dinv = lax.rsqrt(deg)
    dinv_ref[...] = dinv
    u_ref[...] = (x_ref[...] * dinv).astype(jnp.bfloat16)


def _mid_body(agg_ref, u_ref, dinv_ref, w1_ref, b1_ref, w2_ref, z_ref):
    dinv = dinv_ref[...]
    ax = (agg_ref[0].astype(jnp.float32) + agg_ref[1].astype(jnp.float32)
          + u_ref[...].astype(jnp.float32)) * dinv
    h1 = jnp.maximum(
        jnp.dot(ax, w1_ref[...], preferred_element_type=jnp.float32)
        + b1_ref[...], 0.0)
    z2 = jnp.dot(h1, w2_ref[...], preferred_element_type=jnp.float32) * dinv
    z_ref[...] = z2.astype(jnp.bfloat16)


def _final_body(nblk, agg2_ref, z_ref, dinv_ref, batch_ref, b2_ref,
                fc1w_ref, fc1b_ref, fc2w_ref, fc2b_ref, out_ref, acc_ref):
    i = pl.program_id(0)
    r = batch_ref.shape[0]
    agg2 = (agg2_ref[0].astype(jnp.float32)
            + agg2_ref[1].astype(jnp.float32))
    z2 = z_ref[...].astype(jnp.float32)
    out2 = (agg2 + z2) * dinv_ref[...]
    out2a = jnp.concatenate([out2, jnp.ones((r, 1), jnp.float32)], axis=1)
    ids = lax.broadcasted_iota(jnp.int32, (r, NUM_GRAPHS), 1)
    oh = (ids == batch_ref[...]).astype(jnp.float32)
    contrib = lax.dot_general(oh, out2a, (((0,), (0,)), ((), ())),
                              preferred_element_type=jnp.float32)

    @pl.when(i == 0)
    def _():
        acc_ref[...] = contrib

    @pl.when(i > 0)
    def _():
        acc_ref[...] += contrib

    @pl.when(i == nblk - 1)
    def _():
        acc = acc_ref[...]
        g = (acc[:, :32] / jnp.maximum(acc[:, 32:33], 1.0)) + b2_ref[...]
        h = jnp.maximum(
            jnp.dot(g, fc1w_ref[...], preferred_element_type=jnp.float32)
            + fc1b_ref[...], 0.0)
        out_ref[...] = (
            jnp.dot(h, fc2w_ref[...], preferred_element_type=jnp.float32)
            + fc2b_ref[...])


def kernel(x, edge_index, batch, W1, b1, W2, b2, fc1_W, fc1_b, fc2_W, fc2_b):
    n, f = x.shape
    e = edge_index.shape[1]
    nc, ns = 2, 16
    nw = nc * ns

    step = 2 * nw * K
    epad = ((e + step - 1) // step) * step
    pad = epad - e
    eip = jnp.concatenate(
        [edge_index,
         jnp.stack([jnp.zeros((pad,), jnp.int32),
                    jnp.full((pad,), n, jnp.int32)])], axis=1)
    x16 = jnp.pad(x, ((0, 0), (0, 16 - f)))
    w1p = jnp.pad(W1, ((0, 16 - f), (0, 0)))
    nacc = ((n + ns * 128 - 1) // (ns * 128)) * (ns * 128)

    deg2 = _make_deg_kernel(epad, nacc, nc, ns)(eip)
    naccr = ((n + ns * 8 - 1) // (ns * 8)) * (ns * 8)
    deg3 = deg2.reshape(nc, nacc, 1)

    rblk = 5000
    nblk = n // rblk
    dinv, u = pl.pallas_call(
        _prep_body,
        grid=(nblk,),
        in_specs=[
            pl.BlockSpec((nc, rblk, 1), lambda i: (0, i, 0)),
            pl.BlockSpec((rblk, 16), lambda i: (i, 0)),
        ],
        out_specs=[
            pl.BlockSpec((rblk, 1), lambda i: (i, 0)),
            pl.BlockSpec((rblk, 16), lambda i: (i, 0)),
        ],
        out_shape=[
            jax.ShapeDtypeStruct((n, 1), jnp.float32),
            jax.ShapeDtypeStruct((n, 16), jnp.bfloat16),
        ],
    )(deg3, x16)

    agg = _make_edge_agg_kernel(n, epad, nc, ns, 16, jnp.bfloat16, _SHARE0, 1536)(u, eip)

    z = pl.pallas_call(
        _mid_body,
        grid=(nblk,),
        in_specs=[
            pl.BlockSpec((nc, rblk, 16), lambda i: (0, i, 0)),
            pl.BlockSpec((rblk, 16), lambda i: (i, 0)),
            pl.BlockSpec((rblk, 1), lambda i: (i, 0)),
            pl.BlockSpec((16, 64), lambda i: (0, 0)),
            pl.BlockSpec((1, 64), lambda i: (0, 0)),
            pl.BlockSpec((64, 32), lambda i: (0, 0)),
        ],
        out_specs=pl.BlockSpec((rblk, 32), lambda i: (i, 0)),
        out_shape=jax.ShapeDtypeStruct((n, 32), jnp.bfloat16),
    )(agg, u, dinv, w1p, b1.reshape(1, 64), W2)

    agg2 = _make_edge_agg_kernel(n, epad, nc, ns, 32, jnp.bfloat16, _SHARE0, 768)(z, eip)

    out = pl.pallas_call(
        functools.partial(_final_body, nblk),
        grid=(nblk,),
        in_specs=[
            pl.BlockSpec((nc, rblk, 32), lambda i: (0, i, 0)),
            pl.BlockSpec((rblk, 32), lambda i: (i, 0)),
            pl.BlockSpec((rblk, 1), lambda i: (i, 0)),
            pl.BlockSpec((rblk, 1), lambda i: (i, 0)),
            pl.BlockSpec((1, 32), lambda i: (0, 0)),
            pl.BlockSpec((32, 32), lambda i: (0, 0)),
            pl.BlockSpec((1, 32), lambda i: (0, 0)),
            pl.BlockSpec((32, 32), lambda i: (0, 0)),
            pl.BlockSpec((1, 32), lambda i: (0, 0)),
        ],
        out_specs=pl.BlockSpec((NUM_GRAPHS, 32), lambda i: (0, 0)),
        out_shape=jax.ShapeDtypeStruct((NUM_GRAPHS, 32), jnp.float32),
        scratch_shapes=[
            pltpu.VMEM((NUM_GRAPHS, 33), jnp.float32),
        ],
    )(agg2, z, dinv, batch.reshape(n, 1), b2.reshape(1, 32),
      fc1_W, fc1_b.reshape(1, 32), fc2_W, fc2_b.reshape(1, 32))
    return out


# speedup vs baseline: 1.1374x; 1.0080x over previous
import functools

import jax
import jax.numpy as jnp
from jax import lax
from jax.experimental import pallas as pl
from jax.experimental.pallas import tpu as pltpu
from jax.experimental.pallas import tpu_sc as plsc

NUM_GRAPHS = 128
K = 768
_SHARE0 = 0.7


def _fill_rows(ref, nrows, value):
    def body(i, _):
        ref[i, :] = jnp.full((16,), value, jnp.float32)
        return _
    lax.fori_loop(0, nrows, body, None)


def _fill_flat(ref, nvec, value):
    def body(i, _):
        ref[pl.ds(i * 16, 16)] = jnp.full((16,), value, jnp.float32)
        return _
    lax.fori_loop(0, nvec, body, None)


def _zero_shared(acc, zeros_buf, zlen, start, count, align=8):
    done = 0
    while done < count:
        step = min(zlen, count - done)
        pltpu.sync_copy(zeros_buf.at[pl.ds(0, step)],
                        acc.at[pl.ds(pl.multiple_of(start + done, align),
                                     step)])
        done += step


def _make_deg_kernel(epad, nacc, nc, ns):
    nw = nc * ns
    ep = epad // nw
    nchunk = ep // K
    per_tile = nacc // ns
    mesh = plsc.VectorSubcoreMesh(core_axis_name="c", subcore_axis_name="s")

    @functools.partial(
        pl.kernel, mesh=mesh,
        out_type=jax.ShapeDtypeStruct((nc, nacc), jnp.float32),
        scratch_types=[
            pltpu.VMEM((2, K), jnp.int32),
            pltpu.VMEM((K,), jnp.float32),
            pltpu.VMEM((K,), jnp.float32),
            pltpu.VMEM_SHARED((nacc,), jnp.float32),
        ],
        compiler_params=pltpu.CompilerParams(use_tc_tiling_on_sc=False),
    )
    def deg_kernel(ei_hbm, out_hbm, ebuf, ones, zeros, acc):
        c = lax.axis_index("c")
        s = lax.axis_index("s")
        _fill_flat(ones, K // 16, 1.0)
        _fill_flat(zeros, K // 16, 0.0)
        _zero_shared(acc, zeros, K, s * per_tile, per_tile, align=128)
        plsc.subcore_barrier()

        base0 = (c * ns + s) * ep

        def chunk(g, _):
            base = pl.multiple_of(base0 + g * K, 256)
            pltpu.sync_copy(ei_hbm.at[:, pl.ds(base, K)], ebuf)
            pltpu.sync_copy(ones, acc.at[ebuf.at[1]], add=True)
            return _

        lax.fori_loop(0, nchunk, chunk, None)
        plsc.subcore_barrier()
        d0 = pl.multiple_of(s * per_tile, 128)
        pltpu.sync_copy(acc.at[pl.ds(d0, per_tile)],
                        out_hbm.at[c].at[pl.ds(d0, per_tile)])

    return deg_kernel


def _make_edge_agg_kernel(n, epad, nc, ns, feat, dtype, share0, k):
    nacc = ((n + ns * 8 - 1) // (ns * 8)) * (ns * 8)
    per_tile = nacc // ns
    pairs_total = epad // (2 * k * ns)
    npair0 = max(1, min(pairs_total - 1, round(share0 * pairs_total)))
    npair1 = pairs_total - npair0
    ep0 = 2 * k * npair0
    ep1 = 2 * k * npair1
    mesh = plsc.VectorSubcoreMesh(core_axis_name="c", subcore_axis_name="s")

    @functools.partial(
        pl.kernel, mesh=mesh,
        out_type=jax.ShapeDtypeStruct((nc, nacc, feat), dtype),
        scratch_types=[
            pltpu.VMEM((2, 2, k), jnp.int32),
            pltpu.VMEM((2, k, feat), dtype),
            pltpu.VMEM_SHARED((nacc, feat), dtype),
            pltpu.SemaphoreType.DMA,
            pltpu.SemaphoreType.DMA,
            pltpu.SemaphoreType.DMA,
            pltpu.SemaphoreType.DMA,
        ],
        compiler_params=pltpu.CompilerParams(use_tc_tiling_on_sc=False),
    )
    def agg_kernel(tab_hbm, ei_hbm, out_hbm, ebuf, rows, acc,
                   gs0, gs1, is0, is1):
        c = lax.axis_index("c")
        s = lax.axis_index("s")
        gsem = (gs0, gs1)
        isem = (is0, is1)
        def zrow(i, _):
            rows[0, i, :] = jnp.zeros((feat,), dtype)
            return _
        lax.fori_loop(0, k, zrow, None)
        _zero_shared(acc, rows.at[0], k, s * per_tile, per_tile)
        plsc.subcore_barrier()

        npair = jnp.where(c == 0, npair0, npair1)
        base0 = jnp.where(c == 0, s * ep0, ns * ep0 + s * ep1)

        def idx_copy(b, q):
            base = pl.multiple_of(base0 + q * k, 256)
            return pltpu.make_async_copy(
                ei_hbm.at[:, pl.ds(base, k)], ebuf.at[b], isem[b])

        def gather_copy(b):
            return pltpu.make_async_copy(
                tab_hbm.at[ebuf.at[b].at[0]], rows.at[b], gsem[b])

        def scatter(b):
            pltpu.sync_copy(rows.at[b], acc.at[ebuf.at[b].at[1]], add=True)

        idx_copy(0, 0).start()
        idx_copy(0, 0).wait()
        gather_copy(0).start()
        idx_copy(1, 1).start()

        def pair(t, _):
            idx_copy(1, 2 * t + 1).wait()
            gather_copy(1).start()
            gather_copy(0).wait()
            scatter(0)

            @pl.when(t < npair - 1)
            def _():
                idx_copy(0, 2 * t + 2).start()

            gather_copy(1).wait()

            @pl.when(t < npair - 1)
            def _():
                idx_copy(0, 2 * t + 2).wait()
                gather_copy(0).start()

            scatter(1)

            @pl.when(t < npair - 1)
            def _():
                idx_copy(1, 2 * t + 3).start()

            return _

        lax.fori_loop(0, npair, pair, None)
        plsc.subcore_barrier()
        r0 = pl.multiple_of(s * per_tile, 8)
        pltpu.sync_copy(acc.at[pl.ds(r0, per_tile)],
                        out_hbm.at[c].at[pl.ds(r0, per_tile)])

    return agg_kernel


def _prep_body(deg_ref, x_ref, dinv_ref, u_ref):
    deg = deg_ref[0, :, :] + deg_ref[1, :, :] + 1.0
    dinv = lax.rsqrt(deg)
    dinv_ref[...] = dinv
    u_ref[...] = (x_ref[...] * dinv).astype(jnp.bfloat16)


def _mid_body(agg_ref, u_ref, dinv_ref, w1_ref, b1_ref, w2_ref, z_ref):
    dinv = dinv_ref[...]
    ax = (agg_ref[0].astype(jnp.float32) + agg_ref[1].astype(jnp.float32)
          + u_ref[...].astype(jnp.float32)) * dinv
    h1 = jnp.maximum(
        jnp.dot(ax, w1_ref[...], preferred_element_type=jnp.float32)
        + b1_ref[...], 0.0)
    z2 = jnp.dot(h1, w2_ref[...], preferred_element_type=jnp.float32) * dinv
    z_ref[...] = z2.astype(jnp.bfloat16)


def _final_body(nblk, agg2_ref, z_ref, dinv_ref, batch_ref, b2_ref,
                fc1w_ref, fc1b_ref, fc2w_ref, fc2b_ref, out_ref, acc_ref):
    i = pl.program_id(0)
    r = batch_ref.shape[0]
    agg2 = (agg2_ref[0].astype(jnp.float32)
            + agg2_ref[1].astype(jnp.float32))
    z2 = z_ref[...].astype(jnp.float32)
    out2 = (agg2 + z2) * dinv_ref[...]
    out2a = jnp.concatenate([out2, jnp.ones((r, 1), jnp.float32)], axis=1)
    ids = lax.broadcasted_iota(jnp.int32, (r, NUM_GRAPHS), 1)
    oh = (ids == batch_ref[...]).astype(jnp.float32)
    contrib = lax.dot_general(oh, out2a, (((0,), (0,)), ((), ())),
                              preferred_element_type=jnp.float32)

    @pl.when(i == 0)
    def _():
        acc_ref[...] = contrib

    @pl.when(i > 0)
    def _():
        acc_ref[...] += contrib

    @pl.when(i == nblk - 1)
    def _():
        acc = acc_ref[...]
        g = (acc[:, :32] / jnp.maximum(acc[:, 32:33], 1.0)) + b2_ref[...]
        h = jnp.maximum(
            jnp.dot(g, fc1w_ref[...], preferred_element_type=jnp.float32)
            + fc1b_ref[...], 0.0)
        out_ref[...] = (
            jnp.dot(h, fc2w_ref[...], preferred_element_type=jnp.float32)
            + fc2b_ref[...])


def kernel(x, edge_index, batch, W1, b1, W2, b2, fc1_W, fc1_b, fc2_W, fc2_b):
    n, f = x.shape
    e = edge_index.shape[1]
    nc, ns = 2, 16
    nw = nc * ns

    step = 2 * nw * K
    epad = ((e + step - 1) // step) * step
    pad = epad - e
    eip = jnp.concatenate(
        [edge_index,
         jnp.stack([jnp.zeros((pad,), jnp.int32),
                    jnp.full((pad,), n, jnp.int32)])], axis=1)
    x16 = jnp.pad(x, ((0, 0), (0, 16 - f)))
    w1p = jnp.pad(W1, ((0, 16 - f), (0, 0)))
    nacc = ((n + ns * 128 - 1) // (ns * 128)) * (ns * 128)

    deg2 = _make_deg_kernel(epad, nacc, nc, ns)(eip)
    naccr = ((n + ns * 8 - 1) // (ns * 8)) * (ns * 8)
    deg3 = deg2.reshape(nc, nacc, 1)

    rblk = 5000
    nblk = n // rblk
    dinv, u = pl.pallas_call(
        _prep_body,
        grid=(nblk,),
        in_specs=[
            pl.BlockSpec((nc, rblk, 1), lambda i: (0, i, 0)),
            pl.BlockSpec((rblk, 16), lambda i: (i, 0)),
        ],
        out_specs=[
            pl.BlockSpec((rblk, 1), lambda i: (i, 0)),
            pl.BlockSpec((rblk, 16), lambda i: (i, 0)),
        ],
        out_shape=[
            jax.ShapeDtypeStruct((n, 1), jnp.float32),
            jax.ShapeDtypeStruct((n, 16), jnp.bfloat16),
        ],
    )(deg3, x16)

    agg = _make_edge_agg_kernel(n, epad, nc, ns, 16, jnp.bfloat16, _SHARE0, 1536)(u, eip)

    z = pl.pallas_call(
        _mid_body,
        grid=(nblk,),
        in_specs=[
            pl.BlockSpec((nc, rblk, 16), lambda i: (0, i, 0)),
            pl.BlockSpec((rblk, 16), lambda i: (i, 0)),
            pl.BlockSpec((rblk, 1), lambda i: (i, 0)),
            pl.BlockSpec((16, 64), lambda i: (0, 0)),
            pl.BlockSpec((1, 64), lambda i: (0, 0)),
            pl.BlockSpec((64, 32), lambda i: (0, 0)),
        ],
        out_specs=pl.BlockSpec((rblk, 32), lambda i: (i, 0)),
        out_shape=jax.ShapeDtypeStruct((n, 32), jnp.bfloat16),
    )(agg, u, dinv, w1p, b1.reshape(1, 64), W2)

    agg2 = _make_edge_agg_kernel(n, epad, nc, ns, 32, jnp.bfloat16, _SHARE0, 768)(z, eip)

    out = pl.pallas_call(
        functools.partial(_final_body, nblk),
        grid=(nblk,),
        in_specs=[
            pl.BlockSpec((nc, rblk, 32), lambda i: (0, i, 0)),
            pl.BlockSpec((rblk, 32), lambda i: (i, 0)),
            pl.BlockSpec((rblk, 1), lambda i: (i, 0)),
            pl.BlockSpec((rblk, 1), lambda i: (i, 0)),
            pl.BlockSpec((1, 32), lambda i: (0, 0)),
            pl.BlockSpec((32, 32), lambda i: (0, 0)),
            pl.BlockSpec((1, 32), lambda i: (0, 0)),
            pl.BlockSpec((32, 32), lambda i: (0, 0)),
            pl.BlockSpec((1, 32), lambda i: (0, 0)),
        ],
        out_specs=pl.BlockSpec((NUM_GRAPHS, 32), lambda i: (0, 0)),
        out_shape=jax.ShapeDtypeStruct((NUM_GRAPHS, 32), jnp.float32),
        scratch_shapes=[
            pltpu.VMEM((NUM_GRAPHS, 33), jnp.float32),
        ],
    )(agg2, z, dinv, batch.reshape(n, 1), b2.reshape(1, 32),
      fc1_W, fc1_b.reshape(1, 32), fc2_W, fc2_b.reshape(1, 32))
    return out


# speedup vs baseline: 1.1490x; 1.0102x over previous
import functools

import jax
import jax.numpy as jnp
from jax import lax
from jax.experimental import pallas as pl
from jax.experimental.pallas import tpu as pltpu
from jax.experimental.pallas import tpu_sc as plsc

NUM_GRAPHS = 128
K = 768
_SHARE0 = 0.75


def _fill_rows(ref, nrows, value):
    def body(i, _):
        ref[i, :] = jnp.full((16,), value, jnp.float32)
        return _
    lax.fori_loop(0, nrows, body, None)


def _fill_flat(ref, nvec, value):
    def body(i, _):
        ref[pl.ds(i * 16, 16)] = jnp.full((16,), value, jnp.float32)
        return _
    lax.fori_loop(0, nvec, body, None)


def _zero_shared(acc, zeros_buf, zlen, start, count, align=8):
    done = 0
    while done < count:
        step = min(zlen, count - done)
        pltpu.sync_copy(zeros_buf.at[pl.ds(0, step)],
                        acc.at[pl.ds(pl.multiple_of(start + done, align),
                                     step)])
        done += step


def _make_deg_kernel(epad, nacc, nc, ns):
    nw = nc * ns
    ep = epad // nw
    nchunk = ep // K
    per_tile = nacc // ns
    mesh = plsc.VectorSubcoreMesh(core_axis_name="c", subcore_axis_name="s")

    @functools.partial(
        pl.kernel, mesh=mesh,
        out_type=jax.ShapeDtypeStruct((nc, nacc), jnp.float32),
        scratch_types=[
            pltpu.VMEM((2, K), jnp.int32),
            pltpu.VMEM((K,), jnp.float32),
            pltpu.VMEM((K,), jnp.float32),
            pltpu.VMEM_SHARED((nacc,), jnp.float32),
        ],
        compiler_params=pltpu.CompilerParams(use_tc_tiling_on_sc=False),
    )
    def deg_kernel(ei_hbm, out_hbm, ebuf, ones, zeros, acc):
        c = lax.axis_index("c")
        s = lax.axis_index("s")
        _fill_flat(ones, K // 16, 1.0)
        _fill_flat(zeros, K // 16, 0.0)
        _zero_shared(acc, zeros, K, s * per_tile, per_tile, align=128)
        plsc.subcore_barrier()

        base0 = (c * ns + s) * ep

        def chunk(g, _):
            base = pl.multiple_of(base0 + g * K, 256)
            pltpu.sync_copy(ei_hbm.at[:, pl.ds(base, K)], ebuf)
            pltpu.sync_copy(ones, acc.at[ebuf.at[1]], add=True)
            return _

        lax.fori_loop(0, nchunk, chunk, None)
        plsc.subcore_barrier()
        d0 = pl.multiple_of(s * per_tile, 128)
        pltpu.sync_copy(acc.at[pl.ds(d0, per_tile)],
                        out_hbm.at[c].at[pl.ds(d0, per_tile)])

    return deg_kernel


def _make_edge_agg_kernel(n, epad, nc, ns, feat, dtype, share0, k):
    nacc = ((n + ns * 8 - 1) // (ns * 8)) * (ns * 8)
    per_tile = nacc // ns
    pairs_total = epad // (2 * k * ns)
    npair0 = max(1, min(pairs_total - 1, round(share0 * pairs_total)))
    npair1 = pairs_total - npair0
    ep0 = 2 * k * npair0
    ep1 = 2 * k * npair1
    mesh = plsc.VectorSubcoreMesh(core_axis_name="c", subcore_axis_name="s")

    @functools.partial(
        pl.kernel, mesh=mesh,
        out_type=jax.ShapeDtypeStruct((nc, nacc, feat), dtype),
        scratch_types=[
            pltpu.VMEM((2, 2, k), jnp.int32),
            pltpu.VMEM((2, k, feat), dtype),
            pltpu.VMEM_SHARED((nacc, feat), dtype),
            pltpu.SemaphoreType.DMA,
            pltpu.SemaphoreType.DMA,
            pltpu.SemaphoreType.DMA,
            pltpu.SemaphoreType.DMA,
        ],
        compiler_params=pltpu.CompilerParams(use_tc_tiling_on_sc=False),
    )
    def agg_kernel(tab_hbm, ei_hbm, out_hbm, ebuf, rows, acc,
                   gs0, gs1, is0, is1):
        c = lax.axis_index("c")
        s = lax.axis_index("s")
        gsem = (gs0, gs1)
        isem = (is0, is1)
        def zrow(i, _):
            rows[0, i, :] = jnp.zeros((feat,), dtype)
            return _
        lax.fori_loop(0, k, zrow, None)
        _zero_shared(acc, rows.at[0], k, s * per_tile, per_tile)
        plsc.subcore_barrier()

        npair = jnp.where(c == 0, npair0, npair1)
        base0 = jnp.where(c == 0, s * ep0, ns * ep0 + s * ep1)

        def idx_copy(b, q):
            base = pl.multiple_of(base0 + q * k, 256)
            return pltpu.make_async_copy(
                ei_hbm.at[:, pl.ds(base, k)], ebuf.at[b], isem[b])

        def gather_copy(b):
            return pltpu.make_async_copy(
                tab_hbm.at[ebuf.at[b].at[0]], rows.at[b], gsem[b])

        def scatter(b):
            pltpu.sync_copy(rows.at[b], acc.at[ebuf.at[b].at[1]], add=True)

        idx_copy(0, 0).start()
        idx_copy(0, 0).wait()
        gather_copy(0).start()
        idx_copy(1, 1).start()

        def pair(t, _):
            idx_copy(1, 2 * t + 1).wait()
            gather_copy(1).start()
            gather_copy(0).wait()
            scatter(0)

            @pl.when(t < npair - 1)
            def _():
                idx_copy(0, 2 * t + 2).start()

            gather_copy(1).wait()

            @pl.when(t < npair - 1)
            def _():
                idx_copy(0, 2 * t + 2).wait()
                gather_copy(0).start()

            scatter(1)

            @pl.when(t < npair - 1)
            def _():
                idx_copy(1, 2 * t + 3).start()

            return _

        lax.fori_loop(0, npair, pair, None)
        plsc.subcore_barrier()
        r0 = pl.multiple_of(s * per_tile, 8)
        pltpu.sync_copy(acc.at[pl.ds(r0, per_tile)],
                        out_hbm.at[c].at[pl.ds(r0, per_tile)])

    return agg_kernel


def _prep_body(deg_ref, x_ref, dinv_ref, u_ref):
    deg = deg_ref[0, :, :] + deg_ref[1, :, :] + 1.0
    dinv = lax.rsqrt(deg)
    dinv_ref[...] = dinv
    u_ref[...] = (x_ref[...] * dinv).astype(jnp.bfloat16)


def _mid_body(agg_ref, u_ref, dinv_ref, w1_ref, b1_ref, w2_ref, z_ref):
    dinv = dinv_ref[...]
    ax = (agg_ref[0].astype(jnp.float32) + agg_ref[1].astype(jnp.float32)
          + u_ref[...].astype(jnp.float32)) * dinv
    h1 = jnp.maximum(
        jnp.dot(ax, w1_ref[...], preferred_element_type=jnp.float32)
        + b1_ref[...], 0.0)
    z2 = jnp.dot(h1, w2_ref[...], preferred_element_type=jnp.float32) * dinv
    z_ref[...] = z2.astype(jnp.bfloat16)


def _final_body(nblk, agg2_ref, z_ref, dinv_ref, batch_ref, b2_ref,
                fc1w_ref, fc1b_ref, fc2w_ref, fc2b_ref, out_ref, acc_ref):
    i = pl.program_id(0)
    r = batch_ref.shape[0]
    agg2 = (agg2_ref[0].astype(jnp.float32)
            + agg2_ref[1].astype(jnp.float32))
    z2 = z_ref[...].astype(jnp.float32)
    out2 = (agg2 + z2) * dinv_ref[...]
    out2a = jnp.concatenate([out2, jnp.ones((r, 1), jnp.float32)], axis=1)
    ids = lax.broadcasted_iota(jnp.int32, (r, NUM_GRAPHS), 1)
    oh = (ids == batch_ref[...]).astype(jnp.float32)
    contrib = lax.dot_general(oh, out2a, (((0,), (0,)), ((), ())),
                              preferred_element_type=jnp.float32)

    @pl.when(i == 0)
    def _():
        acc_ref[...] = contrib

    @pl.when(i > 0)
    def _():
        acc_ref[...] += contrib

    @pl.when(i == nblk - 1)
    def _():
        acc = acc_ref[...]
        g = (acc[:, :32] / jnp.maximum(acc[:, 32:33], 1.0)) + b2_ref[...]
        h = jnp.maximum(
            jnp.dot(g, fc1w_ref[...], preferred_element_type=jnp.float32)
            + fc1b_ref[...], 0.0)
        out_ref[...] = (
            jnp.dot(h, fc2w_ref[...], preferred_element_type=jnp.float32)
            + fc2b_ref[...])


def kernel(x, edge_index, batch, W1, b1, W2, b2, fc1_W, fc1_b, fc2_W, fc2_b):
    n, f = x.shape
    e = edge_index.shape[1]
    nc, ns = 2, 16
    nw = nc * ns

    step = 2 * nw * K
    epad = ((e + step - 1) // step) * step
    pad = epad - e
    eip = jnp.concatenate(
        [edge_index,
         jnp.stack([jnp.zeros((pad,), jnp.int32),
                    jnp.full((pad,), n, jnp.int32)])], axis=1)
    x16 = jnp.pad(x, ((0, 0), (0, 16 - f)))
    w1p = jnp.pad(W1, ((0, 16 - f), (0, 0)))
    nacc = ((n + ns * 128 - 1) // (ns * 128)) * (ns * 128)

    deg2 = _make_deg_kernel(epad, nacc, nc, ns)(eip)
    naccr = ((n + ns * 8 - 1) // (ns * 8)) * (ns * 8)
    deg3 = deg2.reshape(nc, nacc, 1)

    rblk = 5000
    nblk = n // rblk
    dinv, u = pl.pallas_call(
        _prep_body,
        grid=(nblk,),
        in_specs=[
            pl.BlockSpec((nc, rblk, 1), lambda i: (0, i, 0)),
            pl.BlockSpec((rblk, 16), lambda i: (i, 0)),
        ],
        out_specs=[
            pl.BlockSpec((rblk, 1), lambda i: (i, 0)),
            pl.BlockSpec((rblk, 16), lambda i: (i, 0)),
        ],
        out_shape=[
            jax.ShapeDtypeStruct((n, 1), jnp.float32),
            jax.ShapeDtypeStruct((n, 16), jnp.bfloat16),
        ],
    )(deg3, x16)

    agg = _make_edge_agg_kernel(n, epad, nc, ns, 16, jnp.bfloat16, _SHARE0, 1536)(u, eip)

    z = pl.pallas_call(
        _mid_body,
        grid=(nblk,),
        in_specs=[
            pl.BlockSpec((nc, rblk, 16), lambda i: (0, i, 0)),
            pl.BlockSpec((rblk, 16), lambda i: (i, 0)),
            pl.BlockSpec((rblk, 1), lambda i: (i, 0)),
            pl.BlockSpec((16, 64), lambda i: (0, 0)),
            pl.BlockSpec((1, 64), lambda i: (0, 0)),
            pl.BlockSpec((64, 32), lambda i: (0, 0)),
        ],
        out_specs=pl.BlockSpec((rblk, 32), lambda i: (i, 0)),
        out_shape=jax.ShapeDtypeStruct((n, 32), jnp.bfloat16),
    )(agg, u, dinv, w1p, b1.reshape(1, 64), W2)

    agg2 = _make_edge_agg_kernel(n, epad, nc, ns, 32, jnp.bfloat16, _SHARE0, 768)(z, eip)

    out = pl.pallas_call(
        functools.partial(_final_body, nblk),
        grid=(nblk,),
        in_specs=[
            pl.BlockSpec((nc, rblk, 32), lambda i: (0, i, 0)),
            pl.BlockSpec((rblk, 32), lambda i: (i, 0)),
            pl.BlockSpec((rblk, 1), lambda i: (i, 0)),
            pl.BlockSpec((rblk, 1), lambda i: (i, 0)),
            pl.BlockSpec((1, 32), lambda i: (0, 0)),
            pl.BlockSpec((32, 32), lambda i: (0, 0)),
            pl.BlockSpec((1, 32), lambda i: (0, 0)),
            pl.BlockSpec((32, 32), lambda i: (0, 0)),
            pl.BlockSpec((1, 32), lambda i: (0, 0)),
        ],
        out_specs=pl.BlockSpec((NUM_GRAPHS, 32), lambda i: (0, 0)),
        out_shape=jax.ShapeDtypeStruct((NUM_GRAPHS, 32), jnp.float32),
        scratch_shapes=[
            pltpu.VMEM((NUM_GRAPHS, 33), jnp.float32),
        ],
    )(agg2, z, dinv, batch.reshape(n, 1), b2.reshape(1, 32),
      fc1_W, fc1_b.reshape(1, 32), fc2_W, fc2_b.reshape(1, 32))
    return out


# speedup vs baseline: 1.1564x; 1.0065x over previous
import functools

import jax
import jax.numpy as jnp
from jax import lax
from jax.experimental import pallas as pl
from jax.experimental.pallas import tpu as pltpu
from jax.experimental.pallas import tpu_sc as plsc

NUM_GRAPHS = 128
K = 768
_SHARE0 = 0.8


def _fill_rows(ref, nrows, value):
    def body(i, _):
        ref[i, :] = jnp.full((16,), value, jnp.float32)
        return _
    lax.fori_loop(0, nrows, body, None)


def _fill_flat(ref, nvec, value):
    def body(i, _):
        ref[pl.ds(i * 16, 16)] = jnp.full((16,), value, jnp.float32)
        return _
    lax.fori_loop(0, nvec, body, None)


def _zero_shared(acc, zeros_buf, zlen, start, count, align=8):
    done = 0
    while done < count:
        step = min(zlen, count - done)
        pltpu.sync_copy(zeros_buf.at[pl.ds(0, step)],
                        acc.at[pl.ds(pl.multiple_of(start + done, align),
                                     step)])
        done += step


def _make_deg_kernel(epad, nacc, nc, ns):
    nw = nc * ns
    ep = epad // nw
    nchunk = ep // K
    per_tile = nacc // ns
    mesh = plsc.VectorSubcoreMesh(core_axis_name="c", subcore_axis_name="s")

    @functools.partial(
        pl.kernel, mesh=mesh,
        out_type=jax.ShapeDtypeStruct((nc, nacc), jnp.float32),
        scratch_types=[
            pltpu.VMEM((2, K), jnp.int32),
            pltpu.VMEM((K,), jnp.float32),
            pltpu.VMEM((K,), jnp.float32),
            pltpu.VMEM_SHARED((nacc,), jnp.float32),
        ],
        compiler_params=pltpu.CompilerParams(use_tc_tiling_on_sc=False),
    )
    def deg_kernel(ei_hbm, out_hbm, ebuf, ones, zeros, acc):
        c = lax.axis_index("c")
        s = lax.axis_index("s")
        _fill_flat(ones, K // 16, 1.0)
        _fill_flat(zeros, K // 16, 0.0)
        _zero_shared(acc, zeros, K, s * per_tile, per_tile, align=128)
        plsc.subcore_barrier()

        base0 = (c * ns + s) * ep

        def chunk(g, _):
            base = pl.multiple_of(base0 + g * K, 256)
            pltpu.sync_copy(ei_hbm.at[:, pl.ds(base, K)], ebuf)
            pltpu.sync_copy(ones, acc.at[ebuf.at[1]], add=True)
            return _

        lax.fori_loop(0, nchunk, chunk, None)
        plsc.subcore_barrier()
        d0 = pl.multiple_of(s * per_tile, 128)
        pltpu.sync_copy(acc.at[pl.ds(d0, per_tile)],
                        out_hbm.at[c].at[pl.ds(d0, per_tile)])

    return deg_kernel


def _make_edge_agg_kernel(n, epad, nc, ns, feat, dtype, share0, k):
    nacc = ((n + ns * 8 - 1) // (ns * 8)) * (ns * 8)
    per_tile = nacc // ns
    pairs_total = epad // (2 * k * ns)
    npair0 = max(1, min(pairs_total - 1, round(share0 * pairs_total)))
    npair1 = pairs_total - npair0
    ep0 = 2 * k * npair0
    ep1 = 2 * k * npair1
    mesh = plsc.VectorSubcoreMesh(core_axis_name="c", subcore_axis_name="s")

    @functools.partial(
        pl.kernel, mesh=mesh,
        out_type=jax.ShapeDtypeStruct((nc, nacc, feat), dtype),
        scratch_types=[
            pltpu.VMEM((2, 2, k), jnp.int32),
            pltpu.VMEM((2, k, feat), dtype),
            pltpu.VMEM_SHARED((nacc, feat), dtype),
            pltpu.SemaphoreType.DMA,
            pltpu.SemaphoreType.DMA,
            pltpu.SemaphoreType.DMA,
            pltpu.SemaphoreType.DMA,
        ],
        compiler_params=pltpu.CompilerParams(use_tc_tiling_on_sc=False),
    )
    def agg_kernel(tab_hbm, ei_hbm, out_hbm, ebuf, rows, acc,
                   gs0, gs1, is0, is1):
        c = lax.axis_index("c")
        s = lax.axis_index("s")
        gsem = (gs0, gs1)
        isem = (is0, is1)
        def zrow(i, _):
            rows[0, i, :] = jnp.zeros((feat,), dtype)
            return _
        lax.fori_loop(0, k, zrow, None)
        _zero_shared(acc, rows.at[0], k, s * per_tile, per_tile)
        plsc.subcore_barrier()

        npair = jnp.where(c == 0, npair0, npair1)
        base0 = jnp.where(c == 0, s * ep0, ns * ep0 + s * ep1)

        def idx_copy(b, q):
            base = pl.multiple_of(base0 + q * k, 256)
            return pltpu.make_async_copy(
                ei_hbm.at[:, pl.ds(base, k)], ebuf.at[b], isem[b])

        def gather_copy(b):
            return pltpu.make_async_copy(
                tab_hbm.at[ebuf.at[b].at[0]], rows.at[b], gsem[b])

        def scatter(b):
            pltpu.sync_copy(rows.at[b], acc.at[ebuf.at[b].at[1]], add=True)

        idx_copy(0, 0).start()
        idx_copy(0, 0).wait()
        gather_copy(0).start()
        idx_copy(1, 1).start()

        def pair(t, _):
            idx_copy(1, 2 * t + 1).wait()
            gather_copy(1).start()
            gather_copy(0).wait()
            scatter(0)

            @pl.when(t < npair - 1)
            def _():
                idx_copy(0, 2 * t + 2).start()

            gather_copy(1).wait()

            @pl.when(t < npair - 1)
            def _():
                idx_copy(0, 2 * t + 2).wait()
                gather_copy(0).start()

            scatter(1)

            @pl.when(t < npair - 1)
            def _():
                idx_copy(1, 2 * t + 3).start()

            return _

        lax.fori_loop(0, npair, pair, None)
        plsc.subcore_barrier()
        r0 = pl.multiple_of(s * per_tile, 8)
        pltpu.sync_copy(acc.at[pl.ds(r0, per_tile)],
                        out_hbm.at[c].at[pl.ds(r0, per_tile)])

    return agg_kernel


def _prep_body(deg_ref, x_ref, dinv_ref, u_ref):
    deg = deg_ref[0, :, :] + deg_ref[1, :, :] + 1.0
    dinv = lax.rsqrt(deg)
    dinv_ref[...] = dinv
    u_ref[...] = (x_ref[...] * dinv).astype(jnp.bfloat16)


def _mid_body(agg_ref, u_ref, dinv_ref, w1_ref, b1_ref, w2_ref, z_ref):
    dinv = dinv_ref[...]
    ax = (agg_ref[0].astype(jnp.float32) + agg_ref[1].astype(jnp.float32)
          + u_ref[...].astype(jnp.float32)) * dinv
    h1 = jnp.maximum(
        jnp.dot(ax, w1_ref[...], preferred_element_type=jnp.float32)
        + b1_ref[...], 0.0)
    z2 = jnp.dot(h1, w2_ref[...], preferred_element_type=jnp.float32) * dinv
    z_ref[...] = z2.astype(jnp.bfloat16)


def _final_body(nblk, agg2_ref, z_ref, dinv_ref, batch_ref, b2_ref,
                fc1w_ref, fc1b_ref, fc2w_ref, fc2b_ref, out_ref, acc_ref):
    i = pl.program_id(0)
    r = batch_ref.shape[0]
    agg2 = (agg2_ref[0].astype(jnp.float32)
            + agg2_ref[1].astype(jnp.float32))
    z2 = z_ref[...].astype(jnp.float32)
    out2 = (agg2 + z2) * dinv_ref[...]
    out2a = jnp.concatenate([out2, jnp.ones((r, 1), jnp.float32)], axis=1)
    ids = lax.broadcasted_iota(jnp.int32, (r, NUM_GRAPHS), 1)
    oh = (ids == batch_ref[...]).astype(jnp.float32)
    contrib = lax.dot_general(oh, out2a, (((0,), (0,)), ((), ())),
                              preferred_element_type=jnp.float32)

    @pl.when(i == 0)
    def _():
        acc_ref[...] = contrib

    @pl.when(i > 0)
    def _():
        acc_ref[...] += contrib

    @pl.when(i == nblk - 1)
    def _():
        acc = acc_ref[...]
        g = (acc[:, :32] / jnp.maximum(acc[:, 32:33], 1.0)) + b2_ref[...]
        h = jnp.maximum(
            jnp.dot(g, fc1w_ref[...], preferred_element_type=jnp.float32)
            + fc1b_ref[...], 0.0)
        out_ref[...] = (
            jnp.dot(h, fc2w_ref[...], preferred_element_type=jnp.float32)
            + fc2b_ref[...])


def kernel(x, edge_index, batch, W1, b1, W2, b2, fc1_W, fc1_b, fc2_W, fc2_b):
    n, f = x.shape
    e = edge_index.shape[1]
    nc, ns = 2, 16
    nw = nc * ns

    step = 2 * nw * K
    epad = ((e + step - 1) // step) * step
    pad = epad - e
    eip = jnp.concatenate(
        [edge_index,
         jnp.stack([jnp.zeros((pad,), jnp.int32),
                    jnp.full((pad,), n, jnp.int32)])], axis=1)
    x16 = jnp.pad(x, ((0, 0), (0, 16 - f)))
    w1p = jnp.pad(W1, ((0, 16 - f), (0, 0)))
    nacc = ((n + ns * 128 - 1) // (ns * 128)) * (ns * 128)

    deg2 = _make_deg_kernel(epad, nacc, nc, ns)(eip)
    naccr = ((n + ns * 8 - 1) // (ns * 8)) * (ns * 8)
    deg3 = deg2.reshape(nc, nacc, 1)

    rblk = 5000
    nblk = n // rblk
    dinv, u = pl.pallas_call(
        _prep_body,
        grid=(nblk,),
        in_specs=[
            pl.BlockSpec((nc, rblk, 1), lambda i: (0, i, 0)),
            pl.BlockSpec((rblk, 16), lambda i: (i, 0)),
        ],
        out_specs=[
            pl.BlockSpec((rblk, 1), lambda i: (i, 0)),
            pl.BlockSpec((rblk, 16), lambda i: (i, 0)),
        ],
        out_shape=[
            jax.ShapeDtypeStruct((n, 1), jnp.float32),
            jax.ShapeDtypeStruct((n, 16), jnp.bfloat16),
        ],
    )(deg3, x16)

    agg = _make_edge_agg_kernel(n, epad, nc, ns, 16, jnp.bfloat16, _SHARE0, 1536)(u, eip)

    z = pl.pallas_call(
        _mid_body,
        grid=(nblk,),
        in_specs=[
            pl.BlockSpec((nc, rblk, 16), lambda i: (0, i, 0)),
            pl.BlockSpec((rblk, 16), lambda i: (i, 0)),
            pl.BlockSpec((rblk, 1), lambda i: (i, 0)),
            pl.BlockSpec((16, 64), lambda i: (0, 0)),
            pl.BlockSpec((1, 64), lambda i: (0, 0)),
            pl.BlockSpec((64, 32), lambda i: (0, 0)),
        ],
        out_specs=pl.BlockSpec((rblk, 32), lambda i: (i, 0)),
        out_shape=jax.ShapeDtypeStruct((n, 32), jnp.bfloat16),
    )(agg, u, dinv, w1p, b1.reshape(1, 64), W2)

    agg2 = _make_edge_agg_kernel(n, epad, nc, ns, 32, jnp.bfloat16, _SHARE0, 768)(z, eip)

    out = pl.pallas_call(
        functools.partial(_final_body, nblk),
        grid=(nblk,),
        in_specs=[
            pl.BlockSpec((nc, rblk, 32), lambda i: (0, i, 0)),
            pl.BlockSpec((rblk, 32), lambda i: (i, 0)),
            pl.BlockSpec((rblk, 1), lambda i: (i, 0)),
            pl.BlockSpec((rblk, 1), lambda i: (i, 0)),
            pl.BlockSpec((1, 32), lambda i: (0, 0)),
            pl.BlockSpec((32, 32), lambda i: (0, 0)),
            pl.BlockSpec((1, 32), lambda i: (0, 0)),
            pl.BlockSpec((32, 32), lambda i: (0, 0)),
            pl.BlockSpec((1, 32), lambda i: (0, 0)),
        ],
        out_specs=pl.BlockSpec((NUM_GRAPHS, 32), lambda i: (0, 0)),
        out_shape=jax.ShapeDtypeStruct((NUM_GRAPHS, 32), jnp.float32),
        scratch_shapes=[
            pltpu.VMEM((NUM_GRAPHS, 33), jnp.float32),
        ],
    )(agg2, z, dinv, batch.reshape(n, 1), b2.reshape(1, 32),
      fc1_W, fc1_b.reshape(1, 32), fc2_W, fc2_b.reshape(1, 32))
    return out
